# Initial kernel scaffold; baseline (speedup 1.0000x reference)
import jax
import jax.numpy as jnp
from jax.experimental import pallas as pl


def kernel(v, adj, W1, W2, out_W, out_b):
    raise NotImplementedError("write your pallas kernel here")


# speedup vs baseline: 1.0379x; 1.0379x over previous
import jax
import jax.numpy as jnp
from jax.experimental import pallas as pl
from jax.experimental.pallas import tpu as pltpu

BI = 200


def _gcn_body(v_ref, w1_ref, w2_ref, ow_ref, ob_ref, adj_ref, out_ref,
              u_ref, u2_ref, x_ref):
    p = pl.program_id(0)
    i = pl.program_id(1)
    ni = pl.num_programs(1)
    bi = adj_ref.shape[0]

    @pl.when((p == 0) & (i == 0))
    def _prologue():
        u_ref[...] = jnp.dot(v_ref[...], w1_ref[...],
                             preferred_element_type=jnp.float32)
        x_ref[...] = jnp.zeros_like(x_ref)

    @pl.when(p == 0)
    def _layer1():
        h = jnp.maximum(
            jnp.dot(adj_ref[...], u_ref[...],
                    preferred_element_type=jnp.float32), 0.0)
        u2_ref[pl.ds(i * bi, bi), :] = jnp.dot(
            h, w2_ref[...], preferred_element_type=jnp.float32)

    @pl.when(p == 1)
    def _layer2():
        h = jnp.maximum(
            jnp.dot(adj_ref[...], u2_ref[...],
                    preferred_element_type=jnp.float32), 0.0)
        x_ref[...] += jnp.sum(h, axis=0, keepdims=True)

    @pl.when((p == 1) & (i == ni - 1))
    def _epilogue():
        out_ref[...] = jnp.dot(x_ref[...], ow_ref[...],
                               preferred_element_type=jnp.float32) + ob_ref[...]


def kernel(v, adj, W1, W2, out_W, out_b):
    n, d_in = v.shape
    hid = W2.shape[1]
    label = out_W.shape[1]
    ni = n // BI

    out = pl.pallas_call(
        _gcn_body,
        grid=(2, ni),
        in_specs=[
            pl.BlockSpec((n, d_in), lambda p, i: (0, 0)),
            pl.BlockSpec(W1.shape, lambda p, i: (0, 0)),
            pl.BlockSpec(W2.shape, lambda p, i: (0, 0)),
            pl.BlockSpec(out_W.shape, lambda p, i: (0, 0)),
            pl.BlockSpec((1, label), lambda p, i: (0, 0)),
            pl.BlockSpec((BI, n), lambda p, i: (i, 0)),
        ],
        out_specs=pl.BlockSpec((1, label), lambda p, i: (0, 0)),
        out_shape=jax.ShapeDtypeStruct((1, label), jnp.float32),
        scratch_shapes=[
            pltpu.VMEM((n, W1.shape[1]), jnp.float32),
            pltpu.VMEM((n, hid), jnp.float32),
            pltpu.VMEM((1, W2.shape[1]), jnp.float32),
        ],
        compiler_params=pltpu.CompilerParams(
            dimension_semantics=("arbitrary", "arbitrary"),
        ),
    )(v, W1, W2, out_W, out_b.reshape(1, label), adj)
    return out.reshape(label)


# speedup vs baseline: 1.0621x; 1.0233x over previous
import jax
import jax.numpy as jnp
from jax.experimental import pallas as pl
from jax.experimental.pallas import tpu as pltpu

BI = 400


def _gcn_body(v_ref, w1_ref, w2_ref, ow_ref, ob_ref, adj_ref, out_ref,
              u_ref, u2_ref, x_ref):
    p = pl.program_id(0)
    i = pl.program_id(1)
    ni = pl.num_programs(1)
    bi = adj_ref.shape[0]

    @pl.when((p == 0) & (i == 0))
    def _prologue():
        u_ref[...] = jnp.dot(v_ref[...], w1_ref[...],
                             preferred_element_type=jnp.float32)
        x_ref[...] = jnp.zeros_like(x_ref)

    @pl.when(p == 0)
    def _layer1():
        h = jnp.maximum(
            jnp.dot(adj_ref[...], u_ref[...],
                    preferred_element_type=jnp.float32), 0.0)
        u2_ref[pl.ds(i * bi, bi), :] = jnp.dot(
            h, w2_ref[...], preferred_element_type=jnp.float32)

    @pl.when(p == 1)
    def _layer2():
        h = jnp.maximum(
            jnp.dot(adj_ref[...], u2_ref[...],
                    preferred_element_type=jnp.float32), 0.0)
        x_ref[...] += jnp.sum(h, axis=0, keepdims=True)

    @pl.when((p == 1) & (i == ni - 1))
    def _epilogue():
        out_ref[...] = jnp.dot(x_ref[...], ow_ref[...],
                               preferred_element_type=jnp.float32) + ob_ref[...]


def kernel(v, adj, W1, W2, out_W, out_b):
    n, d_in = v.shape
    hid = W2.shape[1]
    label = out_W.shape[1]
    ni = n // BI

    out = pl.pallas_call(
        _gcn_body,
        grid=(2, ni),
        in_specs=[
            pl.BlockSpec((n, d_in), lambda p, i: (0, 0)),
            pl.BlockSpec(W1.shape, lambda p, i: (0, 0)),
            pl.BlockSpec(W2.shape, lambda p, i: (0, 0)),
            pl.BlockSpec(out_W.shape, lambda p, i: (0, 0)),
            pl.BlockSpec((1, label), lambda p, i: (0, 0)),
            pl.BlockSpec((BI, n), lambda p, i: (i, 0)),
        ],
        out_specs=pl.BlockSpec((1, label), lambda p, i: (0, 0)),
        out_shape=jax.ShapeDtypeStruct((1, label), jnp.float32),
        scratch_shapes=[
            pltpu.VMEM((n, W1.shape[1]), jnp.float32),
            pltpu.VMEM((n, hid), jnp.float32),
            pltpu.VMEM((1, W2.shape[1]), jnp.float32),
        ],
        compiler_params=pltpu.CompilerParams(
            dimension_semantics=("arbitrary", "arbitrary"),
        ),
    )(v, W1, W2, out_W, out_b.reshape(1, label), adj)
    return out.reshape(label)
